# Initial kernel scaffold; baseline (speedup 1.0000x reference)
import jax
import jax.numpy as jnp
from jax.experimental import pallas as pl


def kernel(indices, table, W, b):
    raise NotImplementedError("write your pallas kernel here")


# speedup vs baseline: 2.0887x; 2.0887x over previous
import functools

import jax
import jax.numpy as jnp
from jax import lax
from jax.experimental import pallas as pl
from jax.experimental.pallas import tpu as pltpu
from jax.experimental.pallas import tpu_sc as plsc

_NC = 2
_NS = 16
_NW = _NC * _NS

_TOTAL = 204800
_PER_W = _TOTAL // _NW
_CHUNK = 128
_NCHUNK = _PER_W // _CHUNK

_ROWS_BLK = 5000


def _compress_body(x_ref, w_ref, b_ref, o_ref):
    y = lax.dot_general(x_ref[...], w_ref[...],
                        (((1,), (1,)), ((), ())),
                        preferred_element_type=jnp.float32)
    o_ref[...] = jnp.clip(y + b_ref[...], -1.0, 1.0)


def _compress_table(table, W, b):
    n_rows, pre = table.shape
    comp = W.shape[0]
    grid = n_rows // _ROWS_BLK
    return pl.pallas_call(
        _compress_body,
        grid=(grid,),
        in_specs=[
            pl.BlockSpec((_ROWS_BLK, pre), lambda i: (i, 0)),
            pl.BlockSpec((comp, pre), lambda i: (0, 0)),
            pl.BlockSpec((1, comp), lambda i: (0, 0)),
        ],
        out_specs=pl.BlockSpec((_ROWS_BLK, comp), lambda i: (i, 0)),
        out_shape=jax.ShapeDtypeStruct((n_rows, comp), jnp.float32),
    )(table, W, b.reshape(1, comp))


def _sc_gather_body(comp_hbm, idx_hbm, out_hbm, idx_v, rows_v, sem):
    c = lax.axis_index("c")
    s = lax.axis_index("s")
    wid = s * _NC + c
    pltpu.sync_copy(idx_hbm.at[wid], idx_v)
    base = wid * _PER_W

    def step(j, carry):
        pltpu.async_copy(comp_hbm.at[idx_v.at[j]], rows_v, sem).wait()
        pltpu.sync_copy(rows_v, out_hbm.at[pl.ds(base + j * _CHUNK, _CHUNK)])
        return carry

    lax.fori_loop(0, _NCHUNK, step, 0)


def _sc_gather(comp_table, idx_flat):
    comp = comp_table.shape[1]
    mesh = plsc.VectorSubcoreMesh(core_axis_name="c", subcore_axis_name="s")
    fn = functools.partial(
        pl.kernel,
        out_type=jax.ShapeDtypeStruct((_TOTAL, comp), jnp.float32),
        mesh=mesh,
        scratch_types=[
            pltpu.VMEM((_NCHUNK, _CHUNK), jnp.int32),
            pltpu.VMEM((_CHUNK, comp), jnp.float32),
            pltpu.SemaphoreType.DMA,
        ],
        compiler_params=pltpu.CompilerParams(use_tc_tiling_on_sc=False),
    )(_sc_gather_body)
    return fn(comp_table, idx_flat.reshape(_NW, _NCHUNK, _CHUNK))


def kernel(indices, table, W, b):
    batch, hist = indices.shape
    comp_dim = W.shape[0]
    comp_table = _compress_table(table, W, b)
    idx_flat = indices.reshape(-1).astype(jnp.int32)
    out = _sc_gather(comp_table, idx_flat)
    return out.reshape(batch, hist, comp_dim)


# speedup vs baseline: 3.1232x; 1.4953x over previous
import functools

import jax
import jax.numpy as jnp
from jax import lax
from jax.experimental import pallas as pl
from jax.experimental.pallas import tpu as pltpu
from jax.experimental.pallas import tpu_sc as plsc

_NC = 2
_NS = 16
_NW = _NC * _NS

_BATCH = 4096
_HIST = 50
_ROWS_PER_W = _BATCH // _NW

_ROWS_BLK = 4000


def _compress_body(x_ref, w_ref, b_ref, o_ref):
    y = lax.dot_general(x_ref[...], w_ref[...],
                        (((1,), (1,)), ((), ())),
                        preferred_element_type=jnp.float32)
    o_ref[:, 0:32] = jnp.clip(y + b_ref[...], -1.0, 1.0)


def _compress_table(table, W, b):
    n_rows, pre = table.shape
    comp = W.shape[0]
    grid = n_rows // _ROWS_BLK
    return pl.pallas_call(
        _compress_body,
        grid=(grid,),
        in_specs=[
            pl.BlockSpec((_ROWS_BLK, pre), lambda i: (i, 0)),
            pl.BlockSpec((comp, pre), lambda i: (0, 0)),
            pl.BlockSpec((1, comp), lambda i: (0, 0)),
        ],
        out_specs=pl.BlockSpec((_ROWS_BLK, pre), lambda i: (i, 0)),
        out_shape=jax.ShapeDtypeStruct((n_rows, pre), jnp.float32),
    )(table, W, b.reshape(1, comp))


def _sc_gather_body(comp_hbm, idx_hbm, out_hbm, idx_v, rows_v, sem):
    c = lax.axis_index("c")
    s = lax.axis_index("s")
    wid = s * _NC + c
    pltpu.sync_copy(idx_hbm.at[wid], idx_v)
    base = wid * _ROWS_PER_W

    def step(b, carry):
        pltpu.async_copy(comp_hbm.at[idx_v.at[b]], rows_v, sem).wait()
        pltpu.sync_copy(rows_v, out_hbm.at[base + b])
        return carry

    lax.fori_loop(0, _ROWS_PER_W, step, 0)


def _sc_gather(comp_tbl4, idx_flat):
    comp = comp_tbl4.shape[1]
    mesh = plsc.VectorSubcoreMesh(core_axis_name="c", subcore_axis_name="s")
    fn = functools.partial(
        pl.kernel,
        out_type=jax.ShapeDtypeStruct((_BATCH, _HIST, comp), jnp.float32),
        mesh=mesh,
        scratch_types=[
            pltpu.VMEM((_ROWS_PER_W, _HIST), jnp.int32),
            pltpu.VMEM((_HIST, comp), jnp.float32),
            pltpu.SemaphoreType.DMA,
        ],
        compiler_params=pltpu.CompilerParams(use_tc_tiling_on_sc=False),
    )(_sc_gather_body)
    return fn(comp_tbl4, idx_flat.reshape(_NW, _ROWS_PER_W, _HIST))


def kernel(indices, table, W, b):
    batch, hist = indices.shape
    n_rows, pre = table.shape
    comp_dim = W.shape[0]
    comp_table = _compress_table(table, W, b)
    comp_tbl4 = comp_table.reshape(n_rows * (pre // comp_dim), comp_dim)
    idx_flat = indices.reshape(-1).astype(jnp.int32) * 4
    return _sc_gather(comp_tbl4, idx_flat)


# speedup vs baseline: 4.4422x; 1.4223x over previous
import functools

import jax
import jax.numpy as jnp
from jax import lax
from jax.experimental import pallas as pl
from jax.experimental.pallas import tpu as pltpu
from jax.experimental.pallas import tpu_sc as plsc

_NC = 2
_NS = 16
_NW = _NC * _NS

_BATCH = 4096
_HIST = 50
_ROWS_PER_W = _BATCH // _NW

_ROWS_BLK = 4000


def _compress_body(x_ref, w_ref, b_ref, o_ref):
    y = lax.dot_general(x_ref[...], w_ref[...],
                        (((1,), (1,)), ((), ())),
                        preferred_element_type=jnp.float32)
    o_ref[:, 0:32] = jnp.clip(y + b_ref[...], -1.0, 1.0)


def _compress_table(table, W, b):
    n_rows, pre = table.shape
    comp = W.shape[0]
    grid = n_rows // _ROWS_BLK
    return pl.pallas_call(
        _compress_body,
        grid=(grid,),
        in_specs=[
            pl.BlockSpec((_ROWS_BLK, pre), lambda i: (i, 0)),
            pl.BlockSpec((comp, pre), lambda i: (0, 0)),
            pl.BlockSpec((1, comp), lambda i: (0, 0)),
        ],
        out_specs=pl.BlockSpec((_ROWS_BLK, pre), lambda i: (i, 0)),
        out_shape=jax.ShapeDtypeStruct((n_rows, pre), jnp.float32),
    )(table, W, b.reshape(1, comp))


_GIDX = _HIST
_GPW = _ROWS_PER_W
_GRP = 16
_NGRP = _GPW // _GRP


def _sc_gather_body(comp_hbm, idx_hbm, out_hbm, idx_v, rows_v, gsem, wsem):
    c = lax.axis_index("c")
    s = lax.axis_index("s")
    wid = s * _NC + c
    pltpu.sync_copy(idx_hbm.at[wid], idx_v)
    base = wid * _GPW

    def gather(u, slot):
        return pltpu.make_async_copy(
            comp_hbm.at[idx_v.at[u]],
            rows_v.at[slot, lax.rem(u, _GRP)],
            gsem.at[slot])

    def fire_group(g, slot):
        for k in range(_GRP):
            gather(g * _GRP + k, slot).start()

    def drain_group(g, slot):
        for k in range(_GRP):
            gather(g * _GRP + k, slot).wait()

    def write(g, slot):
        return pltpu.make_async_copy(
            rows_v.at[slot], out_hbm.at[pl.ds(base + g * _GRP, _GRP)],
            wsem.at[slot])

    fire_group(0, 0)

    def step(g, carry):
        slot = lax.rem(g, 2)
        nslot = lax.rem(g + 1, 2)

        @pl.when(g + 1 < _NGRP)
        def _():
            @pl.when(g >= 1)
            def _():
                write(g - 1, nslot).wait()
            fire_group(g + 1, nslot)

        drain_group(g, slot)
        write(g, slot).start()
        return carry

    lax.fori_loop(0, _NGRP, step, 0)
    write(_NGRP - 2, lax.rem(_NGRP - 2, 2)).wait()
    write(_NGRP - 1, lax.rem(_NGRP - 1, 2)).wait()


def _sc_gather(comp_tbl4, idx_flat):
    comp = comp_tbl4.shape[1]
    mesh = plsc.VectorSubcoreMesh(core_axis_name="c", subcore_axis_name="s")
    fn = functools.partial(
        pl.kernel,
        out_type=jax.ShapeDtypeStruct((_BATCH, _HIST, comp), jnp.float32),
        mesh=mesh,
        scratch_types=[
            pltpu.VMEM((_GPW, _GIDX), jnp.int32),
            pltpu.VMEM((2, _GRP, _GIDX, comp), jnp.float32),
            pltpu.SemaphoreType.DMA((2,)),
            pltpu.SemaphoreType.DMA((2,)),
        ],
        compiler_params=pltpu.CompilerParams(use_tc_tiling_on_sc=False),
    )(_sc_gather_body)
    return fn(comp_tbl4, idx_flat.reshape(_NW, _GPW, _GIDX))


def kernel(indices, table, W, b):
    batch, hist = indices.shape
    n_rows, pre = table.shape
    comp_dim = W.shape[0]
    comp_table = _compress_table(table, W, b)
    comp_tbl4 = comp_table.reshape(n_rows * (pre // comp_dim), comp_dim)
    idx_flat = indices.reshape(-1).astype(jnp.int32) * 4
    out = _sc_gather(comp_tbl4, idx_flat)
    return out.reshape(batch, hist, comp_dim)
